# Initial kernel scaffold; baseline (speedup 1.0000x reference)
import jax
import jax.numpy as jnp
from jax.experimental import pallas as pl


def kernel(z_t_mol, z_t_pro, t, molecule_idx, protein_pocket_idx, params):
    raise NotImplementedError("write your pallas kernel here")


# speedup vs baseline: 66.7371x; 66.7371x over previous
import functools

import jax
import jax.numpy as jnp
from jax.experimental import pallas as pl
from jax.experimental.pallas import tpu as pltpu

XD = 3; NA = 16; NR = 20; JD = 64; HID = 128; NL = 4; EED = 8; NG = 32
N_MOL = 1024; N_PRO = 4096; N = N_MOL + N_PRO
CL = 2.0; CP = 1.0; CI = 1.5
NORM = 100.0

TI = 128
TJ = 128
T = N // TI
MOL_TILES_I = N_MOL // TI
MOL_TILES_J = N_MOL // TJ
F = 128


def _pad2(w, rows=F, cols=F, row_off=0):
    out = jnp.zeros((rows, cols), jnp.float32)
    return out.at[row_off:row_off + w.shape[0], :w.shape[1]].set(w)


def _padb(b, cols=F):
    out = jnp.zeros((1, cols), jnp.float32)
    return out.at[0, :b.shape[0]].set(b)


def _silu(x):
    return x * (1.0 / (1.0 + jnp.exp(-x)))


def _embed_kernel(zmf, zpf, x128, idxc, tpad,
                  ae1, ae1b, ae2, ae2b, re1, re1b, re2, re2b,
                  wx, wh, wt, bemb, h0):
    hm = jnp.dot(_silu(jnp.dot(zmf[...], ae1[...],
                               preferred_element_type=jnp.float32) + ae1b[...]),
                 ae2[...], preferred_element_type=jnp.float32) + ae2b[...]
    hp = jnp.dot(_silu(jnp.dot(zpf[...], re1[...],
                               preferred_element_type=jnp.float32) + re1b[...]),
                 re2[...], preferred_element_type=jnp.float32) + re2b[...]
    lane = jax.lax.broadcasted_iota(jnp.int32, (N, F), 1)
    oh = (idxc[...] == lane).astype(jnp.float32)
    ht = jnp.dot(oh, tpad[...], preferred_element_type=jnp.float32)
    base = jnp.dot(x128[...], wx[...], preferred_element_type=jnp.float32) \
        + ht * wt[...] + bemb[...]
    h0[0:N_MOL, :] = base[0:N_MOL, :] + jnp.dot(
        hm, wh[...], preferred_element_type=jnp.float32)
    h0[N_MOL:N, :] = base[N_MOL:N, :] + jnp.dot(
        hp, wh[...], preferred_element_type=jnp.float32)


def _gcl_kernel(bounds, h_ref, x_ref, xT_ref, idxc, idxr, xxc, xxr,
                wsrc, wdst, ce_ref, w2, b2, n1h, n1a, n1b, n2w, n2b,
                hout, A, B):
    it = pl.program_id(0)

    @pl.when(it == 0)
    def _():
        hfull = h_ref[...]
        A[...] = jnp.dot(hfull, wsrc[...], preferred_element_type=jnp.float32)
        B[...] = jnp.dot(hfull, wdst[...], preferred_element_type=jnp.float32)

    ibase = it * TI
    a_i = A[pl.ds(ibase, TI), :]
    x_i = x_ref[pl.ds(ibase, TI), :]
    xx_i = xxc[pl.ds(ibase, TI), :]
    idx_i = idxc[pl.ds(ibase, TI), :]
    i_is_mol = it < MOL_TILES_I

    ce0 = ce_ref[0:1, :]
    ce1 = ce_ref[1:2, :]
    ce2 = ce_ref[2:3, :]

    def j_block(j, acc):
        jbase = j * TJ
        b_j = B[pl.ds(jbase, TJ), :]
        xT_j = xT_ref[:, pl.ds(jbase, TJ)]
        xx_j = xxr[:, pl.ds(jbase, TJ)]
        idx_j = idxr[:, pl.ds(jbase, TJ)]
        j_is_mol = j < MOL_TILES_J
        mm = jnp.logical_and(i_is_mol, j_is_mol)
        pp = jnp.logical_and(jnp.logical_not(i_is_mol),
                             jnp.logical_not(j_is_mol))
        cut2 = jnp.where(mm, CL * CL, jnp.where(pp, CP * CP, CI * CI))
        ce = jnp.where(mm, ce1[...], jnp.where(pp, ce2[...], ce0[...]))
        d2 = xx_i + xx_j - 2.0 * jnp.dot(
            x_i, xT_j, preferred_element_type=jnp.float32)
        mask = jnp.logical_and(idx_i == idx_j, d2 < cut2)
        pre = a_i[:, None, :] + (b_j + ce)[None, :, :]
        u = _silu(pre).reshape(TI * TJ, HID)
        v = jnp.dot(u, w2[...], preferred_element_type=jnp.float32) + b2[...]
        w = _silu(v).reshape(TI, TJ, HID)
        maskf = mask.astype(jnp.float32)
        return acc + jax.lax.dot_general(
            maskf, w, (((1,), (1,)), ((0,), (0,))),
            preferred_element_type=jnp.float32)

    acc = jnp.zeros((TI, HID), jnp.float32)
    acc = jax.lax.fori_loop(bounds[it, 0], bounds[it, 1], j_block, acc)
    acc = jax.lax.fori_loop(bounds[it, 2], bounds[it, 3], j_block, acc)

    agg = acc * (1.0 / NORM)
    h_i = h_ref[pl.ds(ibase, TI), :]
    ni = jnp.dot(h_i, n1h[...], preferred_element_type=jnp.float32) \
        + jnp.dot(agg, n1a[...], preferred_element_type=jnp.float32) + n1b[...]
    nh = jnp.dot(_silu(ni), n2w[...],
                 preferred_element_type=jnp.float32) + n2b[...]
    hout[...] = h_i + nh


def _decode_kernel(h_ref, wout, bout, ad1, ad1b, ad2, ad2b,
                   rd1, rd1b, rd2, rd2b, sdisp, sdm, sdp,
                   em_ref, ep_ref):
    out = jnp.dot(h_ref[...], wout[...],
                  preferred_element_type=jnp.float32) + bout[...]
    om = out[0:N_MOL, :]
    op = out[N_MOL:N, :]
    dm = jnp.dot(_silu(jnp.dot(om, ad1[...],
                               preferred_element_type=jnp.float32) + ad1b[...]),
                 ad2[...], preferred_element_type=jnp.float32) + ad2b[...]
    dp = jnp.dot(_silu(jnp.dot(op, rd1[...],
                               preferred_element_type=jnp.float32) + rd1b[...]),
                 rd2[...], preferred_element_type=jnp.float32) + rd2b[...]
    em_ref[...] = jnp.dot(om, sdisp[...], preferred_element_type=jnp.float32) \
        + jnp.dot(dm, sdm[...], preferred_element_type=jnp.float32)
    ep_ref[...] = jnp.dot(op, sdisp[...], preferred_element_type=jnp.float32) \
        + jnp.dot(dp, sdp[...], preferred_element_type=jnp.float32)


def _full(shape):
    return pl.BlockSpec(shape, lambda *_: tuple(0 for _ in shape))


def kernel(z_t_mol, z_t_pro, t, molecule_idx, protein_pocket_idx, params):
    p = params
    idx_mol = molecule_idx.astype(jnp.int32)
    idx_pro = protein_pocket_idx.astype(jnp.int32)
    idx_joint = jnp.concatenate([idx_mol, idx_pro], 0)

    x3 = jnp.concatenate([z_t_mol[:, :XD], z_t_pro[:, :XD]], 0)
    xx = jnp.sum(x3 * x3, axis=1)
    x128 = jnp.pad(x3, ((0, 0), (0, F - XD)))
    xT = x128.T
    idxc = idx_joint[:, None]
    idxr = idx_joint[None, :]
    xxc = xx[:, None]
    xxr = xx[None, :]

    starts = jnp.arange(T) * TI
    g_lo = idx_joint[starts]
    g_hi = idx_joint[starts + TI - 1]
    mol_lo = jnp.searchsorted(idx_mol, g_lo, side='left').astype(jnp.int32)
    mol_hi = jnp.searchsorted(idx_mol, g_hi, side='right').astype(jnp.int32)
    pro_lo = (N_MOL + jnp.searchsorted(idx_pro, g_lo, side='left')).astype(jnp.int32)
    pro_hi = (N_MOL + jnp.searchsorted(idx_pro, g_hi, side='right')).astype(jnp.int32)
    bounds = jnp.stack([mol_lo // TJ, -(-mol_hi // TJ),
                        pro_lo // TJ, -(-pro_hi // TJ)], axis=1)

    zmf = _pad2(z_t_mol[:, XD:], N_MOL, F)
    zpf = _pad2(z_t_pro[:, XD:], N_PRO, F)
    tpad = _pad2(t, F, 1)
    emb_w, emb_b = p['emb_in']
    embed_in = (zmf, zpf, x128, idxc, tpad,
                _pad2(p['ae1'][0]), _padb(p['ae1'][1]),
                _pad2(p['ae2'][0]), _padb(p['ae2'][1]),
                _pad2(p['re1'][0]), _padb(p['re1'][1]),
                _pad2(p['re2'][0]), _padb(p['re2'][1]),
                _pad2(emb_w[0:XD]), _pad2(emb_w[XD:XD + JD]),
                _padb(emb_w[XD + JD]), _padb(emb_b))
    h = pl.pallas_call(
        _embed_kernel,
        out_shape=jax.ShapeDtypeStruct((N, HID), jnp.float32),
        in_specs=[_full(a.shape) for a in embed_in],
        out_specs=_full((N, HID)),
    )(*embed_in)

    for lp in p['gcl']:
        w1, b1 = lp['e1']
        ce = jnp.dot(p['edge_emb'], w1[2 * HID:]) + b1
        ce = jnp.pad(ce, ((0, 5), (0, 0)))
        n1w, n1b = lp['n1']
        ins = (h, x128, xT, idxc, idxr, xxc, xxr,
               w1[:HID], w1[HID:2 * HID], ce,
               lp['e2'][0], _padb(lp['e2'][1], HID),
               n1w[:HID], n1w[HID:], _padb(n1b, HID),
               lp['n2'][0], _padb(lp['n2'][1], HID))
        grid_spec = pltpu.PrefetchScalarGridSpec(
            num_scalar_prefetch=1,
            grid=(T,),
            in_specs=[_full(a.shape) for a in ins],
            out_specs=pl.BlockSpec((TI, HID), lambda i, *_: (i, 0)),
            scratch_shapes=[pltpu.VMEM((N, HID), jnp.float32),
                            pltpu.VMEM((N, HID), jnp.float32)],
        )
        h = pl.pallas_call(
            _gcl_kernel,
            grid_spec=grid_spec,
            out_shape=jax.ShapeDtypeStruct((N, HID), jnp.float32),
        )(bounds, *ins)

    wout, bout = p['emb_out']
    eye = jnp.eye(F, dtype=jnp.float32)
    sdisp = eye * (jnp.arange(F) < XD).astype(jnp.float32)
    sdm = jnp.zeros((F, F), jnp.float32).at[jnp.arange(NA), XD + jnp.arange(NA)].set(1.0)
    sdp = jnp.zeros((F, F), jnp.float32).at[jnp.arange(NR), XD + jnp.arange(NR)].set(1.0)
    dec_in = (h, _pad2(wout), _padb(bout),
              _pad2(p['ad1'][0], row_off=XD), _padb(p['ad1'][1]),
              _pad2(p['ad2'][0]), _padb(p['ad2'][1]),
              _pad2(p['rd1'][0], row_off=XD), _padb(p['rd1'][1]),
              _pad2(p['rd2'][0]), _padb(p['rd2'][1]),
              sdisp, sdm, sdp)
    em, ep = pl.pallas_call(
        _decode_kernel,
        out_shape=(jax.ShapeDtypeStruct((N_MOL, F), jnp.float32),
                   jax.ShapeDtypeStruct((N_PRO, F), jnp.float32)),
        in_specs=[_full(a.shape) for a in dec_in],
        out_specs=(_full((N_MOL, F)), _full((N_PRO, F))),
    )(*dec_in)

    return (em[:, :XD + NA], ep[:, :XD + NR])
